# baseline (speedup 1.0000x reference)
import functools

import jax
import jax.numpy as jnp
from jax import lax
from jax.experimental import pallas as pl
from jax.experimental.pallas import tpu as pltpu
from jax.experimental.pallas import tpu_sc as plsc

VOCAB = 1000000
D = 64
S = 200
B = 4096
NC = 2
NS = 16
NW = NC * NS
BPW = B // NW
L = 16

CH0, CH1 = 96, 104
assert CH0 + CH1 == S and CH0 % 8 == 0


def _pool_body(ids_hbm, table_hbm, pooled_hbm, idx_v, rows_v, pooled_v,
               sem0, sem1):
    cid = lax.axis_index("c")
    sid = lax.axis_index("s")
    wid = sid * NC + cid
    base = wid * BPW

    pltpu.sync_copy(ids_hbm.at[pl.ds(base * S, BPW * S)], idx_v)

    sems = (sem0, sem1)

    def start(e, buf):
        off = e * S
        pltpu.async_copy(
            table_hbm.at[idx_v.at[pl.ds(off, CH0)]],
            rows_v.at[buf, pl.ds(0, CH0)], sems[buf])
        pltpu.async_copy(
            table_hbm.at[idx_v.at[pl.ds(off + CH0, CH1)]],
            rows_v.at[buf, pl.ds(CH0, CH1)], sems[buf])

    def wait(buf):
        pltpu.make_async_copy(
            table_hbm.at[idx_v.at[pl.ds(0, CH0)]],
            rows_v.at[buf, pl.ds(0, CH0)], sems[buf]).wait()
        pltpu.make_async_copy(
            table_hbm.at[idx_v.at[pl.ds(0, CH1)]],
            rows_v.at[buf, pl.ds(CH0, CH1)], sems[buf]).wait()

    start(0, 0)

    def process(e, buf):
        @pl.when(e + 1 < BPW)
        def _():
            start(e + 1, 1 - buf)

        wait(buf)

        def sum_step(t, acc):
            return tuple(
                acc[c] + rows_v[buf, t, pl.ds(c * L, L)] for c in range(4))

        acc = lax.fori_loop(
            0, S, sum_step, tuple(jnp.zeros((L,), jnp.float32)
                                  for _ in range(4)))
        inv = jnp.float32(1.0 / S)
        for c in range(4):
            pooled_v[e, pl.ds(c * L, L)] = acc[c] * inv

    def outer(e2, _):
        process(e2 * 2, 0)
        process(e2 * 2 + 1, 1)
        return 0

    lax.fori_loop(0, BPW // 2, outer, 0)

    pltpu.sync_copy(pooled_v, pooled_hbm.at[pl.ds(base, BPW)])


@jax.jit
def _pool(ids_flat, table):
    mesh = plsc.VectorSubcoreMesh(core_axis_name="c", subcore_axis_name="s")
    return pl.kernel(
        _pool_body,
        out_type=jax.ShapeDtypeStruct((B, D), jnp.float32),
        mesh=mesh,
        scratch_types=[
            pltpu.VMEM((BPW * S,), jnp.int32),
            pltpu.VMEM((2, S, D), jnp.float32),
            pltpu.VMEM((BPW, D), jnp.float32),
            pltpu.SemaphoreType.DMA,
            pltpu.SemaphoreType.DMA,
        ],
        compiler_params=pltpu.CompilerParams(use_tc_tiling_on_sc=False),
    )(ids_flat, table)


def _mlp_body(x_ref, w1_ref, b1_ref, w2_ref, b2_ref, o_ref):
    x = x_ref[...]
    h = jnp.dot(x, w1_ref[...], preferred_element_type=jnp.float32)
    h = jnp.maximum(h + b1_ref[...], 0.0)
    o_ref[...] = (
        jnp.dot(h, w2_ref[...], preferred_element_type=jnp.float32)
        + b2_ref[...])


@jax.jit
def _mlp(pooled, W1, b1, W2, b2):
    return pl.pallas_call(
        _mlp_body,
        out_shape=jax.ShapeDtypeStruct((B, 2), jnp.float32),
    )(pooled, W1, b1.reshape(1, -1), W2, b2.reshape(1, -1))


def kernel(input_ids, emb_table, W1, b1, W2, b2):
    ids_flat = input_ids.reshape(-1).astype(jnp.int32)
    pooled = _pool(ids_flat, emb_table)
    return _mlp(pooled, W1, b1, W2, b2)


# speedup vs baseline: 1.0461x; 1.0461x over previous
import jax
import jax.numpy as jnp
from jax import lax
from jax.experimental import pallas as pl
from jax.experimental.pallas import tpu as pltpu
from jax.experimental.pallas import tpu_sc as plsc

VOCAB = 1000000
D = 64
S = 200
B = 4096
NC = 2
NS = 16
NW = NC * NS
BPW = B // NW
L = 16
NACC = 4

assert S % NACC == 0


def _pool_body(ids_hbm, table_hbm, pooled_hbm, idx_v, acc_v, pooled_v,
               *sems):
    cid = lax.axis_index("c")
    sid = lax.axis_index("s")
    wid = sid * NC + cid
    base = wid * BPW

    pltpu.sync_copy(ids_hbm.at[:, pl.ds(base, BPW)], idx_v)

    def start(t, r, add):
        pltpu.async_copy(table_hbm.at[idx_v.at[t]], acc_v.at[r], sems[r],
                         add=add)

    def wait_one(r):
        pltpu.make_async_copy(table_hbm.at[idx_v.at[0]], acc_v.at[r],
                              sems[r]).wait()

    for r in range(NACC):
        start(r, r, False)

    def outer(i, _):
        for r in range(NACC):
            wait_one(r)
            start(i * NACC + r, r, True)
        return 0

    lax.fori_loop(1, S // NACC, outer, 0)
    for r in range(NACC):
        wait_one(r)

    inv = jnp.float32(1.0 / S)

    def combine(j, _):
        for c in range(4):
            v = acc_v[0, j, pl.ds(c * L, L)]
            for r in range(1, NACC):
                v = v + acc_v[r, j, pl.ds(c * L, L)]
            pooled_v[j, pl.ds(c * L, L)] = v * inv
        return 0

    lax.fori_loop(0, BPW, combine, 0)

    pltpu.sync_copy(pooled_v, pooled_hbm.at[pl.ds(base, BPW)])


@jax.jit
def _pool(ids_t, table):
    mesh = plsc.VectorSubcoreMesh(core_axis_name="c", subcore_axis_name="s")
    return pl.kernel(
        _pool_body,
        out_type=jax.ShapeDtypeStruct((B, D), jnp.float32),
        mesh=mesh,
        scratch_types=[
            pltpu.VMEM((S, BPW), jnp.int32),
            pltpu.VMEM((NACC, BPW, D), jnp.float32),
            pltpu.VMEM((BPW, D), jnp.float32),
        ] + [pltpu.SemaphoreType.DMA] * NACC,
        compiler_params=pltpu.CompilerParams(use_tc_tiling_on_sc=False),
    )(ids_t, table)


def _mlp_body(x_ref, w1_ref, b1_ref, w2_ref, b2_ref, o_ref):
    x = x_ref[...]
    h = jnp.dot(x, w1_ref[...], preferred_element_type=jnp.float32)
    h = jnp.maximum(h + b1_ref[...], 0.0)
    o_ref[...] = (
        jnp.dot(h, w2_ref[...], preferred_element_type=jnp.float32)
        + b2_ref[...])


@jax.jit
def _mlp(pooled, W1, b1, W2, b2):
    return pl.pallas_call(
        _mlp_body,
        out_shape=jax.ShapeDtypeStruct((B, 2), jnp.float32),
    )(pooled, W1, b1.reshape(1, -1), W2, b2.reshape(1, -1))


def kernel(input_ids, emb_table, W1, b1, W2, b2):
    ids_t = input_ids.astype(jnp.int32).T
    pooled = _pool(ids_t, emb_table)
    return _mlp(pooled, W1, b1, W2, b2)
